# baseline (speedup 1.0000x reference)
import jax
import jax.numpy as jnp
from jax.experimental import pallas as pl
from jax.experimental.pallas import tpu as pltpu

_B, _N, _M = 8, 2048, 2048
_RT = 512
_NI = _N // _RT


def _chamfer_body(p_ref, g_ref, d1_ref, d2_ref):
    i = pl.program_id(1)
    p = p_ref[0]
    g = g_ref[0]
    p2 = jnp.sum(p * p, axis=0, keepdims=True)
    g2 = jnp.sum(g * g, axis=0, keepdims=True)
    lhs = jnp.concatenate([p, p2, jnp.ones_like(p2)], axis=0)
    rhs = jnp.concatenate([-2.0 * g, jnp.ones_like(g2), g2], axis=0)
    d = jax.lax.dot_general(
        lhs, rhs, (((0,), (0,)), ((), ())), preferred_element_type=jnp.float32,
        precision=jax.lax.Precision.HIGHEST,
    )
    d1_ref[0, 0] = jnp.maximum(jnp.min(d, axis=1), 0.0)
    colmin = jnp.min(d, axis=0)

    @pl.when(i == 0)
    def _():
        d2_ref[0, 0] = colmin

    @pl.when(i > 0)
    def _():
        d2_ref[0, 0] = jnp.minimum(d2_ref[0, 0], colmin)


def kernel(pred_points, gt_points):
    p_t = jnp.transpose(pred_points, (0, 2, 1))
    g_t = jnp.transpose(gt_points, (0, 2, 1))
    d1, d2 = pl.pallas_call(
        _chamfer_body,
        grid=(_B, _NI),
        in_specs=[
            pl.BlockSpec((1, 3, _RT), lambda b, i: (b, 0, i)),
            pl.BlockSpec((1, 3, _M), lambda b, i: (b, 0, 0)),
        ],
        out_specs=[
            pl.BlockSpec((1, 1, _RT), lambda b, i: (b, 0, i)),
            pl.BlockSpec((1, 1, _M), lambda b, i: (b, 0, 0)),
        ],
        out_shape=[
            jax.ShapeDtypeStruct((_B, 1, _N), jnp.float32),
            jax.ShapeDtypeStruct((_B, 1, _M), jnp.float32),
        ],
    )(p_t, g_t)
    d2 = jnp.maximum(d2, 0.0)
    return jnp.mean(d1) + jnp.mean(d2)


# speedup vs baseline: 2.7533x; 2.7533x over previous
import jax
import jax.numpy as jnp
from jax.experimental import pallas as pl
from jax.experimental.pallas import tpu as pltpu

_B, _N, _M = 8, 2048, 2048
_RT = 512
_NI = _N // _RT


def _chamfer_body(p_ref, g_ref, d1_ref, d2_ref):
    i = pl.program_id(1)
    p = p_ref[0]
    g = g_ref[0]
    p2 = jnp.sum(p * p, axis=0, keepdims=True)
    g2 = jnp.sum(g * g, axis=0, keepdims=True)
    p2_hi = p2.astype(jnp.bfloat16).astype(jnp.float32)
    p2_lo = p2 - p2_hi
    g2_hi = g2.astype(jnp.bfloat16).astype(jnp.float32)
    g2_lo = g2 - g2_hi
    ones_p = jnp.ones_like(p2)
    ones_g = jnp.ones_like(g2)
    lhs = jnp.concatenate([p, p2_hi, p2_lo, ones_p, ones_p], axis=0)
    rhs = jnp.concatenate(
        [-2.0 * g, ones_g, ones_g, g2_hi, g2_lo], axis=0
    )
    d = jax.lax.dot_general(
        lhs, rhs, (((0,), (0,)), ((), ())), preferred_element_type=jnp.float32
    )
    d1_ref[0, 0] = jnp.maximum(jnp.min(d, axis=1), 0.0)
    colmin = jnp.min(d, axis=0)

    @pl.when(i == 0)
    def _():
        d2_ref[0, 0] = colmin

    @pl.when(i > 0)
    def _():
        d2_ref[0, 0] = jnp.minimum(d2_ref[0, 0], colmin)


def kernel(pred_points, gt_points):
    p_t = jnp.transpose(pred_points, (0, 2, 1))
    g_t = jnp.transpose(gt_points, (0, 2, 1))
    d1, d2 = pl.pallas_call(
        _chamfer_body,
        grid=(_B, _NI),
        in_specs=[
            pl.BlockSpec((1, 3, _RT), lambda b, i: (b, 0, i)),
            pl.BlockSpec((1, 3, _M), lambda b, i: (b, 0, 0)),
        ],
        out_specs=[
            pl.BlockSpec((1, 1, _RT), lambda b, i: (b, 0, i)),
            pl.BlockSpec((1, 1, _M), lambda b, i: (b, 0, 0)),
        ],
        out_shape=[
            jax.ShapeDtypeStruct((_B, 1, _N), jnp.float32),
            jax.ShapeDtypeStruct((_B, 1, _M), jnp.float32),
        ],
    )(p_t, g_t)
    d2 = jnp.maximum(d2, 0.0)
    return jnp.mean(d1) + jnp.mean(d2)
